# baseline (speedup 1.0000x reference)
import functools

import jax
import jax.numpy as jnp
from jax import lax
from jax.experimental import pallas as pl
from jax.experimental.pallas import tpu as pltpu
from jax.experimental.pallas import tpu_sc as plsc

_B = 4096
_S = 200
_H = 128
_T = _B * _S
_BB = 128

_NC = 2
_NS = 16
_NW = _NC * _NS
_PW = _T // _NW
_C = 128
_NCHUNK = _PW // _C


def _prep_body(idx_ref, emb_int_ref, emb_pos_ref, flat_ref, comb_ref):
    i = pl.program_id(0)
    pos = emb_pos_ref[...]

    @pl.when(i == 0)
    def _():
        comb_ref[...] = pos[None, :, :] + emb_int_ref[...][:, None, :]

    flat_ref[...] = idx_ref[...] * _S + lax.broadcasted_iota(jnp.int32, (_BB, _S), 1)


def _prep(interaction, emb_interaction, emb_position):
    return pl.pallas_call(
        _prep_body,
        grid=(_B // _BB,),
        in_specs=[
            pl.BlockSpec((_BB, _S), lambda i: (i, 0)),
            pl.BlockSpec((3, _H), lambda i: (0, 0)),
            pl.BlockSpec((_S, _H), lambda i: (0, 0)),
        ],
        out_specs=[
            pl.BlockSpec((_BB, _S), lambda i: (i, 0)),
            pl.BlockSpec((3, _S, _H), lambda i: (0, 0, 0)),
        ],
        out_shape=[
            jax.ShapeDtypeStruct((_B, _S), jnp.int32),
            jax.ShapeDtypeStruct((3, _S, _H), jnp.float32),
        ],
    )(interaction, emb_interaction, emb_position)


def _sc_body(comb_hbm, flat_hbm, out_hbm, idx_v, rows_v, sem):
    wid = lax.axis_index("s") * _NC + lax.axis_index("c")
    base = wid * _PW

    def chunk(g, carry):
        off = base + g * _C
        pltpu.sync_copy(flat_hbm.at[pl.ds(off, _C)], idx_v)
        pltpu.async_copy(comb_hbm.at[idx_v], rows_v, sem).wait()
        pltpu.sync_copy(rows_v, out_hbm.at[pl.ds(off, _C)])
        return carry

    lax.fori_loop(0, _NCHUNK, chunk, 0)


_sc_gather = functools.partial(
    pl.kernel,
    out_type=jax.ShapeDtypeStruct((_T, _H), jnp.float32),
    mesh=plsc.VectorSubcoreMesh(core_axis_name="c", subcore_axis_name="s"),
    scratch_types=[
        pltpu.VMEM((_C,), jnp.int32),
        pltpu.VMEM((_C, _H), jnp.float32),
        pltpu.SemaphoreType.DMA,
    ],
)(_sc_body)


def kernel(interaction, emb_interaction, emb_position):
    flat, comb3 = _prep(interaction, emb_interaction, emb_position)
    out_flat = _sc_gather(comb3.reshape(3 * _S, _H), flat.reshape(_T))
    return out_flat.reshape(_B, _S, _H)


# speedup vs baseline: 1.1054x; 1.1054x over previous
import functools

import jax
import jax.numpy as jnp
from jax import lax
from jax.experimental import pallas as pl
from jax.experimental.pallas import tpu as pltpu
from jax.experimental.pallas import tpu_sc as plsc

_B = 4096
_S = 200
_H = 128
_T = _B * _S
_BB = 128

_NC = 2
_NS = 16
_NW = _NC * _NS
_PW = _T // _NW
_C = 128
_NCHUNK = _PW // _C


def _prep_body(idx_ref, emb_int_ref, emb_pos_ref, flat_ref, comb_ref):
    i = pl.program_id(0)
    pos = emb_pos_ref[...]

    @pl.when(i == 0)
    def _():
        comb_ref[...] = pos[None, :, :] + emb_int_ref[...][:, None, :]

    flat_ref[...] = idx_ref[...] * _S + lax.broadcasted_iota(jnp.int32, (_BB, _S), 1)


def _prep(interaction, emb_interaction, emb_position):
    return pl.pallas_call(
        _prep_body,
        grid=(_B // _BB,),
        in_specs=[
            pl.BlockSpec((_BB, _S), lambda i: (i, 0)),
            pl.BlockSpec((3, _H), lambda i: (0, 0)),
            pl.BlockSpec((_S, _H), lambda i: (0, 0)),
        ],
        out_specs=[
            pl.BlockSpec((_BB, _S), lambda i: (i, 0)),
            pl.BlockSpec((3, _S, _H), lambda i: (0, 0, 0)),
        ],
        out_shape=[
            jax.ShapeDtypeStruct((_B, _S), jnp.int32),
            jax.ShapeDtypeStruct((3, _S, _H), jnp.float32),
        ],
    )(interaction, emb_interaction, emb_position)


_NBUF = 4


def _sc_body(comb_hbm, flat_hbm, out_hbm, idx_all, *scratch):
    rows = scratch[:_NBUF]
    gsem = scratch[_NBUF : 2 * _NBUF]
    osem = scratch[2 * _NBUF : 3 * _NBUF]
    wid = lax.axis_index("s") * _NC + lax.axis_index("c")
    base = wid * _PW

    pltpu.sync_copy(flat_hbm.at[pl.ds(wid * _NCHUNK, _NCHUNK)], idx_all)

    for b in range(_NBUF):
        pltpu.async_copy(comb_hbm.at[idx_all.at[b]], rows[b], gsem[b])

    def outer(i, carry):
        for b in range(_NBUF):
            g = i * _NBUF + b
            off = base + g * _C
            out_slice = out_hbm.at[pl.ds(off, _C)]
            pltpu.make_async_copy(comb_hbm.at[idx_all.at[g]], rows[b], gsem[b]).wait()
            pltpu.async_copy(rows[b], out_slice, osem[b])
            pltpu.make_async_copy(rows[b], out_slice, osem[b]).wait()

            @pl.when(g + _NBUF < _NCHUNK)
            def _():
                pltpu.async_copy(comb_hbm.at[idx_all.at[g + _NBUF]], rows[b], gsem[b])

        return carry

    lax.fori_loop(0, _NCHUNK // _NBUF, outer, 0)


_sc_gather = functools.partial(
    pl.kernel,
    out_type=jax.ShapeDtypeStruct((_T, _H), jnp.float32),
    mesh=plsc.VectorSubcoreMesh(core_axis_name="c", subcore_axis_name="s"),
    scratch_types=[pltpu.VMEM((_NCHUNK, _C), jnp.int32)]
    + [pltpu.VMEM((_C, _H), jnp.float32) for _ in range(_NBUF)]
    + [pltpu.SemaphoreType.DMA for _ in range(2 * _NBUF)],
)(_sc_body)


def kernel(interaction, emb_interaction, emb_position):
    flat, comb3 = _prep(interaction, emb_interaction, emb_position)
    out_flat = _sc_gather(comb3.reshape(3 * _S, _H), flat.reshape(_T // _C, _C))
    return out_flat.reshape(_B, _S, _H)


# speedup vs baseline: 2.8689x; 2.5953x over previous
import functools

import jax
import jax.numpy as jnp
from jax import lax
from jax.experimental import pallas as pl
from jax.experimental.pallas import tpu as pltpu
from jax.experimental.pallas import tpu_sc as plsc

_B = 4096
_S = 200
_H = 128
_T = _B * _S
_BB = 128

_NC = 2
_NS = 16
_NW = _NC * _NS
_PW = _T // _NW
_C = 128
_NCHUNK = _PW // _C


def _prep_body(idx_ref, emb_int_ref, emb_pos_ref, flat_ref, comb_ref):
    i = pl.program_id(0)
    pos = emb_pos_ref[...]

    @pl.when(i == 0)
    def _():
        comb_ref[...] = pos[None, :, :] + emb_int_ref[...][:, None, :]

    flat_ref[...] = idx_ref[...] * _S + lax.broadcasted_iota(jnp.int32, (_BB, _S), 1)


def _prep(interaction, emb_interaction, emb_position):
    return pl.pallas_call(
        _prep_body,
        grid=(_B // _BB,),
        in_specs=[
            pl.BlockSpec((_BB, _S), lambda i: (i, 0)),
            pl.BlockSpec((3, _H), lambda i: (0, 0)),
            pl.BlockSpec((_S, _H), lambda i: (0, 0)),
        ],
        out_specs=[
            pl.BlockSpec((_BB, _S), lambda i: (i, 0)),
            pl.BlockSpec((3, _S, _H), lambda i: (0, 0, 0)),
        ],
        out_shape=[
            jax.ShapeDtypeStruct((_B, _S), jnp.int32),
            jax.ShapeDtypeStruct((3, _S, _H), jnp.float32),
        ],
    )(interaction, emb_interaction, emb_position)


_NBUF = 4


def _sc_body(comb_hbm, flat_hbm, out_hbm, idx_all, comb_sp, *scratch):
    rows = scratch[:_NBUF]
    gsem = scratch[_NBUF : 2 * _NBUF]
    osem = scratch[2 * _NBUF : 3 * _NBUF]
    sid = lax.axis_index("s")
    wid = sid * _NC + lax.axis_index("c")
    base = wid * _PW

    @pl.when(sid == 0)
    def _():
        pltpu.sync_copy(comb_hbm, comb_sp)

    pltpu.sync_copy(flat_hbm.at[pl.ds(wid * _NCHUNK, _NCHUNK)], idx_all)
    plsc.subcore_barrier()

    for b in range(_NBUF):
        pltpu.async_copy(comb_sp.at[idx_all.at[b]], rows[b], gsem[b])

    def outer(i, carry):
        for b in range(_NBUF):
            g = i * _NBUF + b
            off = base + g * _C
            out_slice = out_hbm.at[pl.ds(off, _C)]
            pltpu.make_async_copy(comb_sp.at[idx_all.at[g]], rows[b], gsem[b]).wait()
            pltpu.async_copy(rows[b], out_slice, osem[b])
            pltpu.make_async_copy(rows[b], out_slice, osem[b]).wait()

            @pl.when(g + _NBUF < _NCHUNK)
            def _():
                pltpu.async_copy(comb_sp.at[idx_all.at[g + _NBUF]], rows[b], gsem[b])

        return carry

    lax.fori_loop(0, _NCHUNK // _NBUF, outer, 0)


_sc_gather = functools.partial(
    pl.kernel,
    out_type=jax.ShapeDtypeStruct((_T, _H), jnp.float32),
    mesh=plsc.VectorSubcoreMesh(core_axis_name="c", subcore_axis_name="s"),
    scratch_types=[
        pltpu.VMEM((_NCHUNK, _C), jnp.int32),
        pltpu.VMEM_SHARED((3 * _S, _H), jnp.float32),
    ]
    + [pltpu.VMEM((_C, _H), jnp.float32) for _ in range(_NBUF)]
    + [pltpu.SemaphoreType.DMA for _ in range(2 * _NBUF)],
)(_sc_body)


def kernel(interaction, emb_interaction, emb_position):
    flat, comb3 = _prep(interaction, emb_interaction, emb_position)
    out_flat = _sc_gather(comb3.reshape(3 * _S, _H), flat.reshape(_T // _C, _C))
    return out_flat.reshape(_B, _S, _H)


# speedup vs baseline: 3.2191x; 1.1220x over previous
import functools

import jax
import jax.numpy as jnp
from jax import lax
from jax.experimental import pallas as pl
from jax.experimental.pallas import tpu as pltpu
from jax.experimental.pallas import tpu_sc as plsc

_B = 4096
_S = 200
_H = 128
_T = _B * _S
_BB = 128

_NC = 2
_NS = 16
_NW = _NC * _NS
_PW = _T // _NW
_C = 128
_NCHUNK = _PW // _C


def _prep_body(idx_ref, emb_int_ref, emb_pos_ref, flat_ref, comb_ref):
    i = pl.program_id(0)
    pos = emb_pos_ref[...]

    @pl.when(i == 0)
    def _():
        comb_ref[...] = pos[None, :, :] + emb_int_ref[...][:, None, :]

    flat_ref[...] = idx_ref[...] * _S + lax.broadcasted_iota(jnp.int32, (_BB, _S), 1)


def _prep(interaction, emb_interaction, emb_position):
    return pl.pallas_call(
        _prep_body,
        grid=(_B // _BB,),
        in_specs=[
            pl.BlockSpec((_BB, _S), lambda i: (i, 0)),
            pl.BlockSpec((3, _H), lambda i: (0, 0)),
            pl.BlockSpec((_S, _H), lambda i: (0, 0)),
        ],
        out_specs=[
            pl.BlockSpec((_BB, _S), lambda i: (i, 0)),
            pl.BlockSpec((3, _S, _H), lambda i: (0, 0, 0)),
        ],
        out_shape=[
            jax.ShapeDtypeStruct((_B, _S), jnp.int32),
            jax.ShapeDtypeStruct((3, _S, _H), jnp.float32),
        ],
    )(interaction, emb_interaction, emb_position)


_NBUF = 5
_LOOK = 2


def _out_slice(out_hbm, base, g):
    return out_hbm.at[pl.ds(base + g * _C, _C)]


def _sc_body(comb_hbm, flat_hbm, out_hbm, idx_all, comb_sp, *scratch):
    rows = scratch[:_NBUF]
    gsem = scratch[_NBUF : 2 * _NBUF]
    osem = scratch[2 * _NBUF : 3 * _NBUF]
    sid = lax.axis_index("s")
    wid = sid * _NC + lax.axis_index("c")
    base = wid * _PW

    @pl.when(sid == 0)
    def _():
        pltpu.sync_copy(comb_hbm, comb_sp)

    pltpu.sync_copy(flat_hbm.at[pl.ds(wid * _NCHUNK, _NCHUNK)], idx_all)
    plsc.subcore_barrier()

    for b in range(_LOOK):
        pltpu.async_copy(comb_sp.at[idx_all.at[b]], rows[b], gsem[b])

    def outer(i, carry):
        for b in range(_NBUF):
            g = i * _NBUF + b
            pltpu.make_async_copy(comb_sp.at[idx_all.at[g]], rows[b], gsem[b]).wait()
            pltpu.async_copy(rows[b], _out_slice(out_hbm, base, g), osem[b])
            b2 = (b + _LOOK) % _NBUF

            @pl.when(g + _LOOK < _NCHUNK)
            def _():
                @pl.when(g + _LOOK >= _NBUF)
                def _():
                    pltpu.make_async_copy(
                        rows[b2], _out_slice(out_hbm, base, g + _LOOK - _NBUF), osem[b2]
                    ).wait()

                pltpu.async_copy(
                    comb_sp.at[idx_all.at[g + _LOOK]], rows[b2], gsem[b2]
                )

        return carry

    lax.fori_loop(0, _NCHUNK // _NBUF, outer, 0)

    for j in range(_NCHUNK - _NBUF, _NCHUNK):
        b = j % _NBUF
        pltpu.make_async_copy(rows[b], _out_slice(out_hbm, base, j), osem[b]).wait()


_sc_gather = functools.partial(
    pl.kernel,
    out_type=jax.ShapeDtypeStruct((_T, _H), jnp.float32),
    mesh=plsc.VectorSubcoreMesh(core_axis_name="c", subcore_axis_name="s"),
    scratch_types=[
        pltpu.VMEM((_NCHUNK, _C), jnp.int32),
        pltpu.VMEM_SHARED((3 * _S, _H), jnp.float32),
    ]
    + [pltpu.VMEM((_C, _H), jnp.float32) for _ in range(_NBUF)]
    + [pltpu.SemaphoreType.DMA for _ in range(2 * _NBUF)],
)(_sc_body)


def kernel(interaction, emb_interaction, emb_position):
    flat, comb3 = _prep(interaction, emb_interaction, emb_position)
    out_flat = _sc_gather(comb3.reshape(3 * _S, _H), flat.reshape(_T // _C, _C))
    return out_flat.reshape(_B, _S, _H)
